# Initial kernel scaffold; baseline (speedup 1.0000x reference)
import jax
import jax.numpy as jnp
from jax.experimental import pallas as pl


def kernel(x, edge_index, edge_attr, private_state, params):
    raise NotImplementedError("write your pallas kernel here")


# speedup vs baseline: 9.2522x; 9.2522x over previous
import functools

import jax
import jax.numpy as jnp
import numpy as np
from jax import lax
from jax.experimental import pallas as pl
from jax.experimental.pallas import tpu as pltpu
from jax.experimental.pallas import tpu_sc as plsc

N = 10000
E = 320000
HID = 256
HEADS = 4
CH = 64
EPS = 1e-5


def _enc_body(x_ref, w_ref, b_ref, g_ref, bb_ref, o_ref):
    h = jnp.dot(x_ref[...], w_ref[...], preferred_element_type=jnp.float32)
    h = h + b_ref[...]
    mu = jnp.mean(h, axis=-1, keepdims=True)
    var = jnp.mean((h - mu) ** 2, axis=-1, keepdims=True)
    h = (h - mu) * lax.rsqrt(var + EPS) * g_ref[...] + bb_ref[...]
    o_ref[...] = jnp.maximum(h, 0.0)


def _encoder(x, w, b, g, bb, blk):
    rows, din = x.shape
    dout = w.shape[1]
    grid = rows // blk
    return pl.pallas_call(
        _enc_body,
        grid=(grid,),
        in_specs=[
            pl.BlockSpec((blk, din), lambda i: (i, 0)),
            pl.BlockSpec((din, dout), lambda i: (0, 0)),
            pl.BlockSpec((1, dout), lambda i: (0, 0)),
            pl.BlockSpec((1, dout), lambda i: (0, 0)),
            pl.BlockSpec((1, dout), lambda i: (0, 0)),
        ],
        out_specs=pl.BlockSpec((blk, dout), lambda i: (i, 0)),
        out_shape=jax.ShapeDtypeStruct((rows, dout), jnp.float32),
    )(x, w, b, g, bb)


def _qkv_body(x_ref, wq, bq, wk, bk, wv, bv, q_ref, k_ref, v_ref):
    xb = x_ref[...]
    q_ref[...] = jnp.dot(xb, wq[...], preferred_element_type=jnp.float32) + bq[...]
    k_ref[...] = jnp.dot(xb, wk[...], preferred_element_type=jnp.float32) + bk[...]
    v_ref[...] = jnp.dot(xb, wv[...], preferred_element_type=jnp.float32) + bv[...]


def _pick(n, pref):
    return pref if n % pref == 0 else n


def _qkv(x, wq, bq, wk, bk, wv, bv):
    rows = x.shape[0]
    blk = _pick(rows, 1024)
    wspec = pl.BlockSpec((HID, HID), lambda i: (0, 0))
    bspec = pl.BlockSpec((1, HID), lambda i: (0, 0))
    ospec = pl.BlockSpec((blk, HID), lambda i: (i, 0))
    oshape = jax.ShapeDtypeStruct((rows, HID), jnp.float32)
    return pl.pallas_call(
        _qkv_body,
        grid=(rows // blk,),
        in_specs=[pl.BlockSpec((blk, HID), lambda i: (i, 0)),
                  wspec, bspec, wspec, bspec, wspec, bspec],
        out_specs=[ospec, ospec, ospec],
        out_shape=[oshape, oshape, oshape],
    )(x, wq, bq, wk, bk, wv, bv)


_HI = jax.lax.Precision.HIGHEST


def _alpha_body(qd_ref, ks_ref, ea_ref, we_ref, bsel_ref, a_ref, m_ref):
    i = pl.program_id(0)
    e = jnp.dot(ea_ref[...], we_ref[...], preferred_element_type=jnp.float32)
    t = qd_ref[...] * (ks_ref[...] + e)
    a = jnp.dot(t, bsel_ref[...], precision=_HI,
                preferred_element_type=jnp.float32)
    a_ref[...] = a
    bm = jnp.max(a, axis=0, keepdims=True)

    @pl.when(i == 0)
    def _():
        m_ref[...] = jnp.full_like(m_ref, -jnp.inf)

    m_ref[...] = jnp.maximum(m_ref[...], bm)


def _alpha(qd, ks, ea, we, bsel):
    blk = _pick(E, 2000)
    return pl.pallas_call(
        _alpha_body,
        grid=(E // blk,),
        in_specs=[
            pl.BlockSpec((blk, HID), lambda i: (i, 0)),
            pl.BlockSpec((blk, HID), lambda i: (i, 0)),
            pl.BlockSpec((blk, HID), lambda i: (i, 0)),
            pl.BlockSpec((HID, HID), lambda i: (0, 0)),
            pl.BlockSpec((HID, 8), lambda i: (0, 0)),
        ],
        out_specs=[pl.BlockSpec((blk, 8), lambda i: (i, 0)),
                   pl.BlockSpec((1, 8), lambda i: (0, 0))],
        out_shape=[jax.ShapeDtypeStruct((E, 8), jnp.float32),
                   jax.ShapeDtypeStruct((1, 8), jnp.float32)],
    )(qd, ks, ea, we, bsel)


def _msg_body(a_ref, m_ref, vs_ref, ea_ref, we_ref, s8_ref, selw_ref,
              msgt_ref, wt_ref):
    w = jnp.exp(a_ref[...] - m_ref[...])
    e = jnp.dot(ea_ref[...], we_ref[...], preferred_element_type=jnp.float32)
    wb = jnp.dot(w, s8_ref[...], precision=_HI,
                 preferred_element_type=jnp.float32)
    m = (vs_ref[...] + e) * wb
    msgt_ref[...] = m.T
    wt_ref[...] = jnp.dot(selw_ref[...], w.T, precision=_HI,
                          preferred_element_type=jnp.float32)


def _msg(alpha, amax, vs, ea, we, s8, selw):
    blk = _pick(E, 2560)
    return pl.pallas_call(
        _msg_body,
        grid=(E // blk,),
        in_specs=[
            pl.BlockSpec((blk, 8), lambda i: (i, 0)),
            pl.BlockSpec((1, 8), lambda i: (0, 0)),
            pl.BlockSpec((blk, HID), lambda i: (i, 0)),
            pl.BlockSpec((blk, HID), lambda i: (i, 0)),
            pl.BlockSpec((HID, HID), lambda i: (0, 0)),
            pl.BlockSpec((8, HID), lambda i: (0, 0)),
            pl.BlockSpec((32, 8), lambda i: (0, 0)),
        ],
        out_specs=[pl.BlockSpec((HID, blk), lambda i: (0, i)),
                   pl.BlockSpec((32, blk), lambda i: (0, i))],
        out_shape=[jax.ShapeDtypeStruct((HID, E), jnp.float32),
                   jax.ShapeDtypeStruct((32, E), jnp.float32)],
    )(alpha, amax, vs, ea, we, s8, selw)


def _update_body(x_ref, numt_ref, dent_ref, sel_ref, ws_ref, bs_ref,
                 g_ref, bb_ref, o_ref):
    i = pl.program_id(0)
    blk = x_ref.shape[0]
    nt = numt_ref[:, pl.ds(i * blk, blk)]
    dt = dent_ref[:, pl.ds(i * blk, blk)]
    recip = 1.0 / jnp.maximum(dt, 1e-30)
    rbt = jnp.dot(sel_ref[...], recip, precision=_HI,
                  preferred_element_type=jnp.float32)
    attn = (nt * rbt).T
    xb = x_ref[...]
    skip = jnp.dot(xb, ws_ref[...], preferred_element_type=jnp.float32) + bs_ref[...]
    h = xb + attn + skip
    mu = jnp.mean(h, axis=-1, keepdims=True)
    var = jnp.mean((h - mu) ** 2, axis=-1, keepdims=True)
    h = (h - mu) * lax.rsqrt(var + EPS) * g_ref[...] + bb_ref[...]
    o_ref[...] = jnp.maximum(h, 0.0)


def _update(x, numt, dent, sel, ws, bs, g, bb):
    rows = x.shape[0]
    blk = _pick(rows, 1024)
    return pl.pallas_call(
        _update_body,
        grid=(rows // blk,),
        in_specs=[
            pl.BlockSpec((blk, HID), lambda i: (i, 0)),
            pl.BlockSpec((HID, rows), lambda i: (0, 0)),
            pl.BlockSpec((32, rows), lambda i: (0, 0)),
            pl.BlockSpec((HID, 32), lambda i: (0, 0)),
            pl.BlockSpec((HID, HID), lambda i: (0, 0)),
            pl.BlockSpec((1, HID), lambda i: (0, 0)),
            pl.BlockSpec((1, HID), lambda i: (0, 0)),
            pl.BlockSpec((1, HID), lambda i: (0, 0)),
        ],
        out_specs=pl.BlockSpec((blk, HID), lambda i: (i, 0)),
        out_shape=jax.ShapeDtypeStruct((rows, HID), jnp.float32),
    )(x, numt, dent, sel, ws, bs, g, bb)


def _ln_row(h, g, b):
    mu = jnp.mean(h, axis=-1, keepdims=True)
    var = jnp.mean((h - mu) ** 2, axis=-1, keepdims=True)
    return (h - mu) * lax.rsqrt(var + EPS) * g + b


def _heads_body(x_ref, ps_ref,
                pw1, pb1, pg1, pbb1, pw2, pb2, pg2, pbb2,
                fwa, fwb, fwc, fb1, fg1, fbb1, fw2, fb2, fg2, fbb2,
                p0w, p0b, p1w, p1b, p2w, p2b,
                v0w, v0b, v1w, v1b, v2w, v2b,
                pol_ref, val_ref, psum, pmax):
    i = pl.program_id(0)
    nb = pl.num_programs(0)
    xb = x_ref[...]

    @pl.when(i == 0)
    def _():
        psum[...] = jnp.zeros_like(psum)
        pmax[...] = jnp.full_like(pmax, -jnp.inf)

    psum[...] += jnp.sum(xb, axis=0, keepdims=True)
    pmax[...] = jnp.maximum(pmax[...], jnp.max(xb, axis=0, keepdims=True))

    @pl.when(i == nb - 1)
    def _():
        gmean = psum[...] * (1.0 / N)
        gmax = pmax[...]
        ps = ps_ref[...]
        pe = jnp.maximum(_ln_row(
            jnp.dot(ps, pw1[...], preferred_element_type=jnp.float32) + pb1[...],
            pg1[...], pbb1[...]), 0.0)
        pe = jnp.maximum(_ln_row(
            jnp.dot(pe, pw2[...], preferred_element_type=jnp.float32) + pb2[...],
            pg2[...], pbb2[...]), 0.0)
        comb = (jnp.dot(gmean, fwa[...], preferred_element_type=jnp.float32)
                + jnp.dot(gmax, fwb[...], preferred_element_type=jnp.float32)
                + jnp.dot(pe, fwc[...], preferred_element_type=jnp.float32)
                + fb1[...])
        fused = jnp.maximum(_ln_row(comb, fg1[...], fbb1[...]), 0.0)
        fused = jnp.maximum(_ln_row(
            jnp.dot(fused, fw2[...], preferred_element_type=jnp.float32) + fb2[...],
            fg2[...], fbb2[...]), 0.0)
        h = jnp.maximum(jnp.dot(fused, p0w[...], preferred_element_type=jnp.float32) + p0b[...], 0.0)
        h = jnp.maximum(jnp.dot(h, p1w[...], preferred_element_type=jnp.float32) + p1b[...], 0.0)
        pol_ref[...] = jnp.dot(h, p2w[...], preferred_element_type=jnp.float32) + p2b[...]
        h = jnp.maximum(jnp.dot(fused, v0w[...], preferred_element_type=jnp.float32) + v0b[...], 0.0)
        h = jnp.maximum(jnp.dot(h, v1w[...], preferred_element_type=jnp.float32) + v1b[...], 0.0)
        val_ref[...] = jnp.tanh(jnp.dot(h, v2w[...], preferred_element_type=jnp.float32) + v2b[...])


def _heads(x, ps, wlist):
    blk = _pick(N, 1000)

    def fullspec(a):
        nd = a.ndim
        return pl.BlockSpec(a.shape, lambda i, _n=nd: (0,) * _n)

    in_specs = [pl.BlockSpec((blk, HID), lambda i: (i, 0)),
                fullspec(ps)] + [fullspec(w) for w in wlist]
    return pl.pallas_call(
        _heads_body,
        grid=(N // blk,),
        in_specs=in_specs,
        out_specs=[pl.BlockSpec((1, 1000), lambda i: (0, 0)),
                   pl.BlockSpec((1, 1), lambda i: (0, 0))],
        out_shape=[jax.ShapeDtypeStruct((1, 1000), jnp.float32),
                   jax.ShapeDtypeStruct((1, 1), jnp.float32)],
        scratch_shapes=[pltpu.VMEM((1, HID), jnp.float32),
                        pltpu.VMEM((1, HID), jnp.float32)],
    )(x, ps, *wlist)


_NW = 32
_GC = 80


def _mesh():
    return plsc.VectorSubcoreMesh(core_axis_name="c", subcore_axis_name="s")


def _sc_gather3(q, k, v, src, dst):
    epw = E // _NW

    @functools.partial(
        pl.kernel,
        mesh=_mesh(),
        out_type=[jax.ShapeDtypeStruct((E, HID), jnp.float32)] * 3,
        scratch_types=[
            pltpu.VMEM((_GC,), jnp.int32),
            pltpu.VMEM((_GC,), jnp.int32),
            pltpu.VMEM((_GC, HID), jnp.float32),
            pltpu.SemaphoreType.DMA,
        ],
    )
    def body(q_hbm, k_hbm, v_hbm, src_hbm, dst_hbm,
             qd_hbm, ks_hbm, vs_hbm, srcv, dstv, rows, sem):
        c = lax.axis_index("c")
        s = lax.axis_index("s")
        wid = s * 2 + c
        base0 = wid * epw

        def step(i, _):
            base = base0 + i * _GC
            pltpu.sync_copy(src_hbm.at[pl.ds(base, _GC)], srcv)
            pltpu.sync_copy(dst_hbm.at[pl.ds(base, _GC)], dstv)
            pltpu.async_copy(q_hbm.at[dstv], rows, sem).wait()
            pltpu.sync_copy(rows, qd_hbm.at[pl.ds(base, _GC)])
            pltpu.async_copy(k_hbm.at[srcv], rows, sem).wait()
            pltpu.sync_copy(rows, ks_hbm.at[pl.ds(base, _GC)])
            pltpu.async_copy(v_hbm.at[srcv], rows, sem).wait()
            pltpu.sync_copy(rows, vs_hbm.at[pl.ds(base, _GC)])
            return 0

        lax.fori_loop(0, epw // _GC, step, 0)

    return body(q, k, v, src, dst)


def _sc_scatter(msgT, wT, dst):
    npad = 10240
    EC = 640
    iters = E // EC

    @functools.partial(
        pl.kernel,
        mesh=_mesh(),
        out_type=[jax.ShapeDtypeStruct((HID, npad), jnp.float32),
                  jax.ShapeDtypeStruct((32, npad), jnp.float32)],
        scratch_types=[
            pltpu.VMEM((EC,), jnp.int32),
            pltpu.VMEM((8, EC), jnp.float32),
            pltpu.VMEM((8, npad), jnp.float32),
            pltpu.VMEM((npad,), jnp.float32),
        ],
        compiler_params=pltpu.CompilerParams(needs_layout_passes=False),
    )
    def body(msgT_hbm, wT_hbm, dst_hbm, numT_hbm, denT_hbm,
             dstv, mbuf, acc, dacc):
        c = lax.axis_index("c")
        s = lax.axis_index("s")
        t = s * 2 + c
        zeros = jnp.zeros((16,), jnp.float32)

        def zstep(i, _):
            for r in range(8):
                acc[r, pl.ds(i * 16, 16)] = zeros
            dacc[pl.ds(i * 16, 16)] = zeros
            return 0

        lax.fori_loop(0, npad // 16, zstep, 0)

        def step(i, _):
            base = i * EC
            pltpu.sync_copy(dst_hbm.at[pl.ds(base, EC)], dstv)
            pltpu.sync_copy(msgT_hbm.at[pl.ds(t * 8, 8), pl.ds(base, EC)],
                            mbuf)
            for g in range(EC // 16):
                dvec = dstv[pl.ds(g * 16, 16)]
                for r in range(8):
                    vals = mbuf[r, pl.ds(g * 16, 16)]
                    plsc.addupdate_scatter(
                        acc, [jnp.full((16,), r, jnp.int32), dvec], vals)
            return 0

        lax.fori_loop(0, iters, step, 0)
        pltpu.sync_copy(acc, numT_hbm.at[pl.ds(t * 8, 8)])

        def dstep(i, _):
            base = i * EC
            pltpu.sync_copy(dst_hbm.at[pl.ds(base, EC)], dstv)
            pltpu.sync_copy(wT_hbm.at[pl.ds(t * 8, 8), pl.ds(base, EC)], mbuf)
            for g in range(EC // 16):
                dvec = dstv[pl.ds(g * 16, 16)]
                vals = mbuf[0, pl.ds(g * 16, 16)]
                plsc.addupdate_scatter(dacc, [dvec], vals)
            return 0

        lax.fori_loop(0, jnp.where(t < 4, iters, 0), dstep, 0)

        def dwb(i, _):
            acc[0, pl.ds(i * 16, 16)] = dacc[pl.ds(i * 16, 16)]
            for r in range(1, 8):
                acc[r, pl.ds(i * 16, 16)] = zeros
            return 0

        lax.fori_loop(0, jnp.where(t < 4, npad // 16, 0), dwb, 0)

        def dwb2(i, _):
            pltpu.sync_copy(acc, denT_hbm.at[pl.ds(t * 8, 8)])
            return 0

        lax.fori_loop(0, jnp.where(t < 4, 1, 0), dwb2, 0)
        return None

    return body(msgT, wT, dst)


def kernel(x, edge_index, edge_attr, private_state, params):
    f32 = jnp.float32
    src = edge_index[0].astype(jnp.int32)
    dst = edge_index[1].astype(jnp.int32)

    eye4 = jnp.eye(4, dtype=f32)
    bsel = jnp.zeros((HID, 8), f32).at[:, :4].set(
        jnp.repeat(eye4, CH, axis=0) * (1.0 / np.sqrt(CH)))
    s8 = jnp.zeros((8, HID), f32).at[:4, :].set(jnp.repeat(eye4, CH, axis=1))
    selw = jnp.zeros((32, 8), f32).at[jnp.arange(4) * 8, jnp.arange(4)].set(1.0)
    sel = jnp.zeros((HID, 32), f32).at[jnp.arange(HID),
                                       (jnp.arange(HID) // CH) * 8].set(1.0)

    def r2(a):
        return a.reshape(1, -1)

    ne, ee = params["node_enc"], params["edge_enc"]
    xh = _encoder(x, ne["lin"]["W"], r2(ne["lin"]["b"]),
                  r2(ne["ln"]["g"]), r2(ne["ln"]["b"]), blk=_pick(N, 1000))
    if N == 10000:
        xh = jnp.zeros((10240, HID), f32).at[:N].set(xh)
    ea = _encoder(edge_attr, ee["lin"]["W"], r2(ee["lin"]["b"]),
                  r2(ee["ln"]["g"]), r2(ee["ln"]["b"]), blk=_pick(E, 2000))

    for lp in params["gnn"]:
        q, k, v = _qkv(xh, lp["q"]["W"], r2(lp["q"]["b"]),
                       lp["k"]["W"], r2(lp["k"]["b"]),
                       lp["v"]["W"], r2(lp["v"]["b"]))
        qd, ks, vs = _sc_gather3(q, k, v, src, dst)
        alpha, amax = _alpha(qd, ks, ea, lp["e"]["W"], bsel)
        msgT, wT = _msg(alpha, amax, vs, ea, lp["e"]["W"], s8, selw)
        numT, denT = _sc_scatter(msgT, wT, dst)
        xh = _update(xh, numT, denT, sel, lp["skip"]["W"], r2(lp["skip"]["b"]),
                     r2(lp["ln"]["g"]), r2(lp["ln"]["b"]))

    pp, fp = params["priv"], params["fusion"]
    fwa = fp["lin1"]["W"][:HID]
    fwb = fp["lin1"]["W"][HID:2 * HID]
    fwc = fp["lin1"]["W"][2 * HID:]
    wlist = [
        pp["lin1"]["W"], r2(pp["lin1"]["b"]), r2(pp["ln1"]["g"]), r2(pp["ln1"]["b"]),
        pp["lin2"]["W"], r2(pp["lin2"]["b"]), r2(pp["ln2"]["g"]), r2(pp["ln2"]["b"]),
        fwa, fwb, fwc, r2(fp["lin1"]["b"]), r2(fp["ln1"]["g"]), r2(fp["ln1"]["b"]),
        fp["lin2"]["W"], r2(fp["lin2"]["b"]), r2(fp["ln2"]["g"]), r2(fp["ln2"]["b"]),
        params["policy"][0]["W"], r2(params["policy"][0]["b"]),
        params["policy"][1]["W"], r2(params["policy"][1]["b"]),
        params["policy"][2]["W"], r2(params["policy"][2]["b"]),
        params["value"][0]["W"], r2(params["value"][0]["b"]),
        params["value"][1]["W"], r2(params["value"][1]["b"]),
        params["value"][2]["W"], r2(params["value"][2]["b"]),
    ]
    pol, val = _heads(xh[:N], r2(private_state), wlist)
    return pol, val


# speedup vs baseline: 9.6960x; 1.0480x over previous
import functools

import jax
import jax.numpy as jnp
import numpy as np
from jax import lax
from jax.experimental import pallas as pl
from jax.experimental.pallas import tpu as pltpu
from jax.experimental.pallas import tpu_sc as plsc

N = 10000
E = 320000
HID = 256
HEADS = 4
CH = 64
EPS = 1e-5


def _enc_body(x_ref, w_ref, b_ref, g_ref, bb_ref, o_ref):
    h = jnp.dot(x_ref[...], w_ref[...], preferred_element_type=jnp.float32)
    h = h + b_ref[...]
    mu = jnp.mean(h, axis=-1, keepdims=True)
    var = jnp.mean((h - mu) ** 2, axis=-1, keepdims=True)
    h = (h - mu) * lax.rsqrt(var + EPS) * g_ref[...] + bb_ref[...]
    o_ref[...] = jnp.maximum(h, 0.0)


def _encoder(x, w, b, g, bb, blk):
    rows, din = x.shape
    dout = w.shape[1]
    grid = rows // blk
    return pl.pallas_call(
        _enc_body,
        grid=(grid,),
        in_specs=[
            pl.BlockSpec((blk, din), lambda i: (i, 0)),
            pl.BlockSpec((din, dout), lambda i: (0, 0)),
            pl.BlockSpec((1, dout), lambda i: (0, 0)),
            pl.BlockSpec((1, dout), lambda i: (0, 0)),
            pl.BlockSpec((1, dout), lambda i: (0, 0)),
        ],
        out_specs=pl.BlockSpec((blk, dout), lambda i: (i, 0)),
        out_shape=jax.ShapeDtypeStruct((rows, dout), jnp.float32),
    )(x, w, b, g, bb)


def _qkv_body(x_ref, wq, bq, wkv, bkv, q_ref, kv_ref):
    xb = x_ref[...]
    q_ref[...] = jnp.dot(xb, wq[...], preferred_element_type=jnp.float32) + bq[...]
    kv_ref[...] = jnp.dot(xb, wkv[...], preferred_element_type=jnp.float32) + bkv[...]


def _pick(n, pref):
    return pref if n % pref == 0 else n


def _qkv(x, wq, bq, wkv, bkv):
    rows = x.shape[0]
    blk = _pick(rows, 1024)
    return pl.pallas_call(
        _qkv_body,
        grid=(rows // blk,),
        in_specs=[pl.BlockSpec((blk, HID), lambda i: (i, 0)),
                  pl.BlockSpec((HID, HID), lambda i: (0, 0)),
                  pl.BlockSpec((1, HID), lambda i: (0, 0)),
                  pl.BlockSpec((HID, 2 * HID), lambda i: (0, 0)),
                  pl.BlockSpec((1, 2 * HID), lambda i: (0, 0))],
        out_specs=[pl.BlockSpec((blk, HID), lambda i: (i, 0)),
                   pl.BlockSpec((blk, 2 * HID), lambda i: (i, 0))],
        out_shape=[jax.ShapeDtypeStruct((rows, HID), jnp.float32),
                   jax.ShapeDtypeStruct((rows, 2 * HID), jnp.float32)],
    )(x, wq, bq, wkv, bkv)


_HI = jax.lax.Precision.HIGHEST


def _alpha_body(qd_ref, ks_ref, ea_ref, we_ref, bsel_ref, a_ref, m_ref):
    i = pl.program_id(0)
    e = jnp.dot(ea_ref[...], we_ref[...], preferred_element_type=jnp.float32)
    t = qd_ref[...] * (ks_ref[...] + e)
    a = jnp.dot(t, bsel_ref[...], precision=_HI,
                preferred_element_type=jnp.float32)
    a_ref[...] = a
    bm = jnp.max(a, axis=0, keepdims=True)

    @pl.when(i == 0)
    def _():
        m_ref[...] = jnp.full_like(m_ref, -jnp.inf)

    m_ref[...] = jnp.maximum(m_ref[...], bm)


def _alpha(qd, ks, ea, we, bsel):
    blk = _pick(E, 2000)
    return pl.pallas_call(
        _alpha_body,
        grid=(E // blk,),
        in_specs=[
            pl.BlockSpec((blk, HID), lambda i: (i, 0)),
            pl.BlockSpec((blk, HID), lambda i: (i, 0)),
            pl.BlockSpec((blk, HID), lambda i: (i, 0)),
            pl.BlockSpec((HID, HID), lambda i: (0, 0)),
            pl.BlockSpec((HID, 8), lambda i: (0, 0)),
        ],
        out_specs=[pl.BlockSpec((blk, 8), lambda i: (i, 0)),
                   pl.BlockSpec((1, 8), lambda i: (0, 0))],
        out_shape=[jax.ShapeDtypeStruct((E, 8), jnp.float32),
                   jax.ShapeDtypeStruct((1, 8), jnp.float32)],
    )(qd, ks, ea, we, bsel)


def _msg_body(a_ref, m_ref, vs_ref, ea_ref, we_ref, s8_ref, selw_ref,
              msgt_ref, wt_ref):
    w = jnp.exp(a_ref[...] - m_ref[...])
    e = jnp.dot(ea_ref[...], we_ref[...], preferred_element_type=jnp.float32)
    wb = jnp.dot(w, s8_ref[...], precision=_HI,
                 preferred_element_type=jnp.float32)
    m = (vs_ref[...] + e) * wb
    msgt_ref[...] = m.T
    wt_ref[...] = jnp.dot(selw_ref[...], w.T, precision=_HI,
                          preferred_element_type=jnp.float32)


def _msg(alpha, amax, vs, ea, we, s8, selw):
    blk = _pick(E, 2560)
    return pl.pallas_call(
        _msg_body,
        grid=(E // blk,),
        in_specs=[
            pl.BlockSpec((blk, 8), lambda i: (i, 0)),
            pl.BlockSpec((1, 8), lambda i: (0, 0)),
            pl.BlockSpec((blk, HID), lambda i: (i, 1)),
            pl.BlockSpec((blk, HID), lambda i: (i, 0)),
            pl.BlockSpec((HID, HID), lambda i: (0, 0)),
            pl.BlockSpec((8, HID), lambda i: (0, 0)),
            pl.BlockSpec((32, 8), lambda i: (0, 0)),
        ],
        out_specs=[pl.BlockSpec((HID, blk), lambda i: (0, i)),
                   pl.BlockSpec((32, blk), lambda i: (0, i))],
        out_shape=[jax.ShapeDtypeStruct((HID, E), jnp.float32),
                   jax.ShapeDtypeStruct((32, E), jnp.float32)],
    )(alpha, amax, vs, ea, we, s8, selw)


def _update_body(x_ref, numt_ref, dent_ref, sel_ref, ws_ref, bs_ref,
                 g_ref, bb_ref, o_ref):
    i = pl.program_id(0)
    blk = x_ref.shape[0]
    nt = numt_ref[:, pl.ds(i * blk, blk)]
    dt = dent_ref[:, pl.ds(i * blk, blk)]
    recip = 1.0 / jnp.maximum(dt, 1e-30)
    rbt = jnp.dot(sel_ref[...], recip, precision=_HI,
                  preferred_element_type=jnp.float32)
    attn = (nt * rbt).T
    xb = x_ref[...]
    skip = jnp.dot(xb, ws_ref[...], preferred_element_type=jnp.float32) + bs_ref[...]
    h = xb + attn + skip
    mu = jnp.mean(h, axis=-1, keepdims=True)
    var = jnp.mean((h - mu) ** 2, axis=-1, keepdims=True)
    h = (h - mu) * lax.rsqrt(var + EPS) * g_ref[...] + bb_ref[...]
    o_ref[...] = jnp.maximum(h, 0.0)


def _update(x, numt, dent, sel, ws, bs, g, bb):
    rows = x.shape[0]
    blk = _pick(rows, 1024)
    return pl.pallas_call(
        _update_body,
        grid=(rows // blk,),
        in_specs=[
            pl.BlockSpec((blk, HID), lambda i: (i, 0)),
            pl.BlockSpec((HID, rows), lambda i: (0, 0)),
            pl.BlockSpec((32, rows), lambda i: (0, 0)),
            pl.BlockSpec((HID, 32), lambda i: (0, 0)),
            pl.BlockSpec((HID, HID), lambda i: (0, 0)),
            pl.BlockSpec((1, HID), lambda i: (0, 0)),
            pl.BlockSpec((1, HID), lambda i: (0, 0)),
            pl.BlockSpec((1, HID), lambda i: (0, 0)),
        ],
        out_specs=pl.BlockSpec((blk, HID), lambda i: (i, 0)),
        out_shape=jax.ShapeDtypeStruct((rows, HID), jnp.float32),
    )(x, numt, dent, sel, ws, bs, g, bb)


def _ln_row(h, g, b):
    mu = jnp.mean(h, axis=-1, keepdims=True)
    var = jnp.mean((h - mu) ** 2, axis=-1, keepdims=True)
    return (h - mu) * lax.rsqrt(var + EPS) * g + b


def _heads_body(x_ref, ps_ref,
                pw1, pb1, pg1, pbb1, pw2, pb2, pg2, pbb2,
                fwa, fwb, fwc, fb1, fg1, fbb1, fw2, fb2, fg2, fbb2,
                p0w, p0b, p1w, p1b, p2w, p2b,
                v0w, v0b, v1w, v1b, v2w, v2b,
                pol_ref, val_ref, psum, pmax):
    i = pl.program_id(0)
    nb = pl.num_programs(0)
    xb = x_ref[...]

    @pl.when(i == 0)
    def _():
        psum[...] = jnp.zeros_like(psum)
        pmax[...] = jnp.full_like(pmax, -jnp.inf)

    psum[...] += jnp.sum(xb, axis=0, keepdims=True)
    pmax[...] = jnp.maximum(pmax[...], jnp.max(xb, axis=0, keepdims=True))

    @pl.when(i == nb - 1)
    def _():
        gmean = psum[...] * (1.0 / N)
        gmax = pmax[...]
        ps = ps_ref[...]
        pe = jnp.maximum(_ln_row(
            jnp.dot(ps, pw1[...], preferred_element_type=jnp.float32) + pb1[...],
            pg1[...], pbb1[...]), 0.0)
        pe = jnp.maximum(_ln_row(
            jnp.dot(pe, pw2[...], preferred_element_type=jnp.float32) + pb2[...],
            pg2[...], pbb2[...]), 0.0)
        comb = (jnp.dot(gmean, fwa[...], preferred_element_type=jnp.float32)
                + jnp.dot(gmax, fwb[...], preferred_element_type=jnp.float32)
                + jnp.dot(pe, fwc[...], preferred_element_type=jnp.float32)
                + fb1[...])
        fused = jnp.maximum(_ln_row(comb, fg1[...], fbb1[...]), 0.0)
        fused = jnp.maximum(_ln_row(
            jnp.dot(fused, fw2[...], preferred_element_type=jnp.float32) + fb2[...],
            fg2[...], fbb2[...]), 0.0)
        h = jnp.maximum(jnp.dot(fused, p0w[...], preferred_element_type=jnp.float32) + p0b[...], 0.0)
        h = jnp.maximum(jnp.dot(h, p1w[...], preferred_element_type=jnp.float32) + p1b[...], 0.0)
        pol_ref[...] = jnp.dot(h, p2w[...], preferred_element_type=jnp.float32) + p2b[...]
        h = jnp.maximum(jnp.dot(fused, v0w[...], preferred_element_type=jnp.float32) + v0b[...], 0.0)
        h = jnp.maximum(jnp.dot(h, v1w[...], preferred_element_type=jnp.float32) + v1b[...], 0.0)
        val_ref[...] = jnp.tanh(jnp.dot(h, v2w[...], preferred_element_type=jnp.float32) + v2b[...])


def _heads(x, ps, wlist):
    blk = _pick(N, 1000)

    def fullspec(a):
        nd = a.ndim
        return pl.BlockSpec(a.shape, lambda i, _n=nd: (0,) * _n)

    in_specs = [pl.BlockSpec((blk, HID), lambda i: (i, 0)),
                fullspec(ps)] + [fullspec(w) for w in wlist]
    return pl.pallas_call(
        _heads_body,
        grid=(N // blk,),
        in_specs=in_specs,
        out_specs=[pl.BlockSpec((1, 1000), lambda i: (0, 0)),
                   pl.BlockSpec((1, 1), lambda i: (0, 0))],
        out_shape=[jax.ShapeDtypeStruct((1, 1000), jnp.float32),
                   jax.ShapeDtypeStruct((1, 1), jnp.float32)],
        scratch_shapes=[pltpu.VMEM((1, HID), jnp.float32),
                        pltpu.VMEM((1, HID), jnp.float32)],
    )(x, ps, *wlist)


_NW = 32
_GC = 80


def _mesh():
    return plsc.VectorSubcoreMesh(core_axis_name="c", subcore_axis_name="s")


def _sc_gather2(q, kv, src, dst):
    epw = E // _NW

    @functools.partial(
        pl.kernel,
        mesh=_mesh(),
        out_type=[jax.ShapeDtypeStruct((E, HID), jnp.float32),
                  jax.ShapeDtypeStruct((E, 2 * HID), jnp.float32)],
        scratch_types=[
            pltpu.VMEM((_GC,), jnp.int32),
            pltpu.VMEM((_GC,), jnp.int32),
            pltpu.VMEM((_GC, HID), jnp.float32),
            pltpu.VMEM((_GC, 2 * HID), jnp.float32),
            pltpu.SemaphoreType.DMA,
        ],
    )
    def body(q_hbm, kv_hbm, src_hbm, dst_hbm,
             qd_hbm, kvs_hbm, srcv, dstv, qbuf, kvbuf, sem):
        c = lax.axis_index("c")
        s = lax.axis_index("s")
        wid = s * 2 + c
        base0 = wid * epw

        def step(i, _):
            base = base0 + i * _GC
            pltpu.sync_copy(src_hbm.at[pl.ds(base, _GC)], srcv)
            pltpu.sync_copy(dst_hbm.at[pl.ds(base, _GC)], dstv)
            d1 = pltpu.async_copy(q_hbm.at[dstv], qbuf, sem)
            d2 = pltpu.async_copy(kv_hbm.at[srcv], kvbuf, sem)
            d1.wait()
            d2.wait()
            pltpu.sync_copy(qbuf, qd_hbm.at[pl.ds(base, _GC)])
            pltpu.sync_copy(kvbuf, kvs_hbm.at[pl.ds(base, _GC)])
            return 0

        lax.fori_loop(0, epw // _GC, step, 0)

    return body(q, kv, src, dst)


def _sc_scatter(msgT, wT, dst):
    npad = 10240
    EC = 640
    iters = E // EC

    @functools.partial(
        pl.kernel,
        mesh=_mesh(),
        out_type=[jax.ShapeDtypeStruct((HID, npad), jnp.float32),
                  jax.ShapeDtypeStruct((32, npad), jnp.float32)],
        scratch_types=[
            pltpu.VMEM((EC,), jnp.int32),
            pltpu.VMEM((8, EC), jnp.float32),
            pltpu.VMEM((8, npad), jnp.float32),
            pltpu.VMEM((npad,), jnp.float32),
        ],
        compiler_params=pltpu.CompilerParams(needs_layout_passes=False),
    )
    def body(msgT_hbm, wT_hbm, dst_hbm, numT_hbm, denT_hbm,
             dstv, mbuf, acc, dacc):
        c = lax.axis_index("c")
        s = lax.axis_index("s")
        t = s * 2 + c
        zeros = jnp.zeros((16,), jnp.float32)

        def zstep(i, _):
            for r in range(8):
                acc[r, pl.ds(i * 16, 16)] = zeros
            dacc[pl.ds(i * 16, 16)] = zeros
            return 0

        lax.fori_loop(0, npad // 16, zstep, 0)

        def step(i, _):
            base = i * EC
            pltpu.sync_copy(dst_hbm.at[pl.ds(base, EC)], dstv)
            pltpu.sync_copy(msgT_hbm.at[pl.ds(t * 8, 8), pl.ds(base, EC)],
                            mbuf)
            for g in range(EC // 16):
                dvec = dstv[pl.ds(g * 16, 16)]
                for r in range(8):
                    vals = mbuf[r, pl.ds(g * 16, 16)]
                    plsc.addupdate_scatter(
                        acc, [jnp.full((16,), r, jnp.int32), dvec], vals)
            return 0

        lax.fori_loop(0, iters, step, 0)
        pltpu.sync_copy(acc, numT_hbm.at[pl.ds(t * 8, 8)])

        def dstep(i, _):
            base = i * EC
            pltpu.sync_copy(dst_hbm.at[pl.ds(base, EC)], dstv)
            pltpu.sync_copy(wT_hbm.at[pl.ds(t * 8, 8), pl.ds(base, EC)], mbuf)
            for g in range(EC // 16):
                dvec = dstv[pl.ds(g * 16, 16)]
                vals = mbuf[0, pl.ds(g * 16, 16)]
                plsc.addupdate_scatter(dacc, [dvec], vals)
            return 0

        lax.fori_loop(0, jnp.where(t < 4, iters, 0), dstep, 0)

        def dwb(i, _):
            acc[0, pl.ds(i * 16, 16)] = dacc[pl.ds(i * 16, 16)]
            for r in range(1, 8):
                acc[r, pl.ds(i * 16, 16)] = zeros
            return 0

        lax.fori_loop(0, jnp.where(t < 4, npad // 16, 0), dwb, 0)

        def dwb2(i, _):
            pltpu.sync_copy(acc, denT_hbm.at[pl.ds(t * 8, 8)])
            return 0

        lax.fori_loop(0, jnp.where(t < 4, 1, 0), dwb2, 0)
        return None

    return body(msgT, wT, dst)


def kernel(x, edge_index, edge_attr, private_state, params):
    f32 = jnp.float32
    src = edge_index[0].astype(jnp.int32)
    dst = edge_index[1].astype(jnp.int32)

    eye4 = jnp.eye(4, dtype=f32)
    bsel = jnp.zeros((HID, 8), f32).at[:, :4].set(
        jnp.repeat(eye4, CH, axis=0) * (1.0 / np.sqrt(CH)))
    s8 = jnp.zeros((8, HID), f32).at[:4, :].set(jnp.repeat(eye4, CH, axis=1))
    selw = jnp.zeros((32, 8), f32).at[jnp.arange(4) * 8, jnp.arange(4)].set(1.0)
    sel = jnp.zeros((HID, 32), f32).at[jnp.arange(HID),
                                       (jnp.arange(HID) // CH) * 8].set(1.0)

    def r2(a):
        return a.reshape(1, -1)

    ne, ee = params["node_enc"], params["edge_enc"]
    xh = _encoder(x, ne["lin"]["W"], r2(ne["lin"]["b"]),
                  r2(ne["ln"]["g"]), r2(ne["ln"]["b"]), blk=_pick(N, 1000))
    if N == 10000:
        xh = jnp.zeros((10240, HID), f32).at[:N].set(xh)
    ea = _encoder(edge_attr, ee["lin"]["W"], r2(ee["lin"]["b"]),
                  r2(ee["ln"]["g"]), r2(ee["ln"]["b"]), blk=_pick(E, 2000))

    for lp in params["gnn"]:
        wkv = jnp.concatenate([lp["k"]["W"], lp["v"]["W"]], axis=1)
        bkv = jnp.concatenate([lp["k"]["b"], lp["v"]["b"]]).reshape(1, -1)
        q, kv = _qkv(xh, lp["q"]["W"], r2(lp["q"]["b"]), wkv, bkv)
        qd, kvs = _sc_gather2(q, kv, src, dst)
        alpha, amax = _alpha(qd, kvs, ea, lp["e"]["W"], bsel)
        msgT, wT = _msg(alpha, amax, kvs, ea, lp["e"]["W"], s8, selw)
        numT, denT = _sc_scatter(msgT, wT, dst)
        xh = _update(xh, numT, denT, sel, lp["skip"]["W"], r2(lp["skip"]["b"]),
                     r2(lp["ln"]["g"]), r2(lp["ln"]["b"]))

    pp, fp = params["priv"], params["fusion"]
    fwa = fp["lin1"]["W"][:HID]
    fwb = fp["lin1"]["W"][HID:2 * HID]
    fwc = fp["lin1"]["W"][2 * HID:]
    wlist = [
        pp["lin1"]["W"], r2(pp["lin1"]["b"]), r2(pp["ln1"]["g"]), r2(pp["ln1"]["b"]),
        pp["lin2"]["W"], r2(pp["lin2"]["b"]), r2(pp["ln2"]["g"]), r2(pp["ln2"]["b"]),
        fwa, fwb, fwc, r2(fp["lin1"]["b"]), r2(fp["ln1"]["g"]), r2(fp["ln1"]["b"]),
        fp["lin2"]["W"], r2(fp["lin2"]["b"]), r2(fp["ln2"]["g"]), r2(fp["ln2"]["b"]),
        params["policy"][0]["W"], r2(params["policy"][0]["b"]),
        params["policy"][1]["W"], r2(params["policy"][1]["b"]),
        params["policy"][2]["W"], r2(params["policy"][2]["b"]),
        params["value"][0]["W"], r2(params["value"][0]["b"]),
        params["value"][1]["W"], r2(params["value"][1]["b"]),
        params["value"][2]["W"], r2(params["value"][2]["b"]),
    ]
    pol, val = _heads(xh[:N], r2(private_state), wlist)
    return pol, val


# speedup vs baseline: 11.8611x; 1.2233x over previous
import functools

import jax
import jax.numpy as jnp
import numpy as np
from jax import lax
from jax.experimental import pallas as pl
from jax.experimental.pallas import tpu as pltpu
from jax.experimental.pallas import tpu_sc as plsc

N = 10000
E = 320000
HID = 256
HEADS = 4
CH = 64
EPS = 1e-5


def _enc_body(x_ref, w_ref, b_ref, g_ref, bb_ref, o_ref):
    h = jnp.dot(x_ref[...], w_ref[...], preferred_element_type=jnp.float32)
    h = h + b_ref[...]
    mu = jnp.mean(h, axis=-1, keepdims=True)
    var = jnp.mean((h - mu) ** 2, axis=-1, keepdims=True)
    h = (h - mu) * lax.rsqrt(var + EPS) * g_ref[...] + bb_ref[...]
    o_ref[...] = jnp.maximum(h, 0.0)


def _encoder(x, w, b, g, bb, blk):
    rows, din = x.shape
    dout = w.shape[1]
    grid = rows // blk
    return pl.pallas_call(
        _enc_body,
        grid=(grid,),
        in_specs=[
            pl.BlockSpec((blk, din), lambda i: (i, 0)),
            pl.BlockSpec((din, dout), lambda i: (0, 0)),
            pl.BlockSpec((1, dout), lambda i: (0, 0)),
            pl.BlockSpec((1, dout), lambda i: (0, 0)),
            pl.BlockSpec((1, dout), lambda i: (0, 0)),
        ],
        out_specs=pl.BlockSpec((blk, dout), lambda i: (i, 0)),
        out_shape=jax.ShapeDtypeStruct((rows, dout), jnp.float32),
    )(x, w, b, g, bb)


def _qkv_body(x_ref, wq, bq, wkv, bkv, q_ref, kv_ref):
    xb = x_ref[...]
    q_ref[...] = jnp.dot(xb, wq[...], preferred_element_type=jnp.float32) + bq[...]
    kv_ref[...] = jnp.dot(xb, wkv[...], preferred_element_type=jnp.float32) + bkv[...]


def _pick(n, pref):
    return pref if n % pref == 0 else n


def _qkv(x, wq, bq, wkv, bkv):
    rows = x.shape[0]
    blk = _pick(rows, 1024)
    return pl.pallas_call(
        _qkv_body,
        grid=(rows // blk,),
        in_specs=[pl.BlockSpec((blk, HID), lambda i: (i, 0)),
                  pl.BlockSpec((HID, HID), lambda i: (0, 0)),
                  pl.BlockSpec((1, HID), lambda i: (0, 0)),
                  pl.BlockSpec((HID, 2 * HID), lambda i: (0, 0)),
                  pl.BlockSpec((1, 2 * HID), lambda i: (0, 0))],
        out_specs=[pl.BlockSpec((blk, HID), lambda i: (i, 0)),
                   pl.BlockSpec((blk, 2 * HID), lambda i: (i, 0))],
        out_shape=[jax.ShapeDtypeStruct((rows, HID), jnp.float32),
                   jax.ShapeDtypeStruct((rows, 2 * HID), jnp.float32)],
    )(x, wq, bq, wkv, bkv)


_HI = jax.lax.Precision.HIGHEST


def _alpha_body(qd_ref, ks_ref, ea_ref, we_ref, bsel_ref, a_ref, m_ref):
    i = pl.program_id(0)
    e = jnp.dot(ea_ref[...], we_ref[...], preferred_element_type=jnp.float32)
    t = qd_ref[...] * (ks_ref[...] + e)
    a = jnp.dot(t, bsel_ref[...], precision=_HI,
                preferred_element_type=jnp.float32)
    a_ref[...] = a
    bm = jnp.max(a, axis=0, keepdims=True)

    @pl.when(i == 0)
    def _():
        m_ref[...] = jnp.full_like(m_ref, -jnp.inf)

    m_ref[...] = jnp.maximum(m_ref[...], bm)


def _alpha(qd, ks, ea, we, bsel):
    blk = _pick(E, 2000)
    return pl.pallas_call(
        _alpha_body,
        grid=(E // blk,),
        in_specs=[
            pl.BlockSpec((blk, HID), lambda i: (i, 0)),
            pl.BlockSpec((blk, HID), lambda i: (i, 0)),
            pl.BlockSpec((blk, HID), lambda i: (i, 0)),
            pl.BlockSpec((HID, HID), lambda i: (0, 0)),
            pl.BlockSpec((HID, 8), lambda i: (0, 0)),
        ],
        out_specs=[pl.BlockSpec((blk, 8), lambda i: (i, 0)),
                   pl.BlockSpec((1, 8), lambda i: (0, 0))],
        out_shape=[jax.ShapeDtypeStruct((E, 8), jnp.float32),
                   jax.ShapeDtypeStruct((1, 8), jnp.float32)],
    )(qd, ks, ea, we, bsel)


def _msg_body(a_ref, m_ref, vs_ref, ea_ref, we_ref, s8_ref, selw_ref,
              msgt_ref, wt_ref):
    w = jnp.exp(a_ref[...] - m_ref[...])
    e = jnp.dot(ea_ref[...], we_ref[...], preferred_element_type=jnp.float32)
    wb = jnp.dot(w, s8_ref[...], precision=_HI,
                 preferred_element_type=jnp.float32)
    m = (vs_ref[...] + e) * wb
    msgt_ref[...] = m.T
    wt_ref[...] = jnp.dot(selw_ref[...], w.T, precision=_HI,
                          preferred_element_type=jnp.float32)


def _msg(alpha, amax, vs, ea, we, s8, selw):
    blk = _pick(E, 2560)
    return pl.pallas_call(
        _msg_body,
        grid=(E // blk,),
        in_specs=[
            pl.BlockSpec((blk, 8), lambda i: (i, 0)),
            pl.BlockSpec((1, 8), lambda i: (0, 0)),
            pl.BlockSpec((blk, HID), lambda i: (i, 1)),
            pl.BlockSpec((blk, HID), lambda i: (i, 0)),
            pl.BlockSpec((HID, HID), lambda i: (0, 0)),
            pl.BlockSpec((8, HID), lambda i: (0, 0)),
            pl.BlockSpec((32, 8), lambda i: (0, 0)),
        ],
        out_specs=[pl.BlockSpec((HID, blk), lambda i: (0, i)),
                   pl.BlockSpec((32, blk), lambda i: (0, i))],
        out_shape=[jax.ShapeDtypeStruct((HID, E), jnp.float32),
                   jax.ShapeDtypeStruct((32, E), jnp.float32)],
    )(alpha, amax, vs, ea, we, s8, selw)


def _update_body(x_ref, numt_ref, dent_ref, sel_ref, ws_ref, bs_ref,
                 g_ref, bb_ref, o_ref):
    i = pl.program_id(0)
    blk = x_ref.shape[0]
    nt = numt_ref[:, pl.ds(i * blk, blk)]
    dt = dent_ref[:, pl.ds(i * blk, blk)]
    recip = 1.0 / jnp.maximum(dt, 1e-30)
    rbt = jnp.dot(sel_ref[...], recip, precision=_HI,
                  preferred_element_type=jnp.float32)
    attn = (nt * rbt).T
    xb = x_ref[...]
    skip = jnp.dot(xb, ws_ref[...], preferred_element_type=jnp.float32) + bs_ref[...]
    h = xb + attn + skip
    mu = jnp.mean(h, axis=-1, keepdims=True)
    var = jnp.mean((h - mu) ** 2, axis=-1, keepdims=True)
    h = (h - mu) * lax.rsqrt(var + EPS) * g_ref[...] + bb_ref[...]
    o_ref[...] = jnp.maximum(h, 0.0)


def _update(x, numt, dent, sel, ws, bs, g, bb):
    rows = x.shape[0]
    blk = _pick(rows, 1024)
    return pl.pallas_call(
        _update_body,
        grid=(rows // blk,),
        in_specs=[
            pl.BlockSpec((blk, HID), lambda i: (i, 0)),
            pl.BlockSpec((HID, rows), lambda i: (0, 0)),
            pl.BlockSpec((32, rows), lambda i: (0, 0)),
            pl.BlockSpec((HID, 32), lambda i: (0, 0)),
            pl.BlockSpec((HID, HID), lambda i: (0, 0)),
            pl.BlockSpec((1, HID), lambda i: (0, 0)),
            pl.BlockSpec((1, HID), lambda i: (0, 0)),
            pl.BlockSpec((1, HID), lambda i: (0, 0)),
        ],
        out_specs=pl.BlockSpec((blk, HID), lambda i: (i, 0)),
        out_shape=jax.ShapeDtypeStruct((rows, HID), jnp.float32),
    )(x, numt, dent, sel, ws, bs, g, bb)


def _ln_row(h, g, b):
    mu = jnp.mean(h, axis=-1, keepdims=True)
    var = jnp.mean((h - mu) ** 2, axis=-1, keepdims=True)
    return (h - mu) * lax.rsqrt(var + EPS) * g + b


def _heads_body(x_ref, ps_ref,
                pw1, pb1, pg1, pbb1, pw2, pb2, pg2, pbb2,
                fwa, fwb, fwc, fb1, fg1, fbb1, fw2, fb2, fg2, fbb2,
                p0w, p0b, p1w, p1b, p2w, p2b,
                v0w, v0b, v1w, v1b, v2w, v2b,
                pol_ref, val_ref, psum, pmax):
    i = pl.program_id(0)
    nb = pl.num_programs(0)
    xb = x_ref[...]

    @pl.when(i == 0)
    def _():
        psum[...] = jnp.zeros_like(psum)
        pmax[...] = jnp.full_like(pmax, -jnp.inf)

    psum[...] += jnp.sum(xb, axis=0, keepdims=True)
    pmax[...] = jnp.maximum(pmax[...], jnp.max(xb, axis=0, keepdims=True))

    @pl.when(i == nb - 1)
    def _():
        gmean = psum[...] * (1.0 / N)
        gmax = pmax[...]
        ps = ps_ref[...]
        pe = jnp.maximum(_ln_row(
            jnp.dot(ps, pw1[...], preferred_element_type=jnp.float32) + pb1[...],
            pg1[...], pbb1[...]), 0.0)
        pe = jnp.maximum(_ln_row(
            jnp.dot(pe, pw2[...], preferred_element_type=jnp.float32) + pb2[...],
            pg2[...], pbb2[...]), 0.0)
        comb = (jnp.dot(gmean, fwa[...], preferred_element_type=jnp.float32)
                + jnp.dot(gmax, fwb[...], preferred_element_type=jnp.float32)
                + jnp.dot(pe, fwc[...], preferred_element_type=jnp.float32)
                + fb1[...])
        fused = jnp.maximum(_ln_row(comb, fg1[...], fbb1[...]), 0.0)
        fused = jnp.maximum(_ln_row(
            jnp.dot(fused, fw2[...], preferred_element_type=jnp.float32) + fb2[...],
            fg2[...], fbb2[...]), 0.0)
        h = jnp.maximum(jnp.dot(fused, p0w[...], preferred_element_type=jnp.float32) + p0b[...], 0.0)
        h = jnp.maximum(jnp.dot(h, p1w[...], preferred_element_type=jnp.float32) + p1b[...], 0.0)
        pol_ref[...] = jnp.dot(h, p2w[...], preferred_element_type=jnp.float32) + p2b[...]
        h = jnp.maximum(jnp.dot(fused, v0w[...], preferred_element_type=jnp.float32) + v0b[...], 0.0)
        h = jnp.maximum(jnp.dot(h, v1w[...], preferred_element_type=jnp.float32) + v1b[...], 0.0)
        val_ref[...] = jnp.tanh(jnp.dot(h, v2w[...], preferred_element_type=jnp.float32) + v2b[...])


def _heads(x, ps, wlist):
    blk = _pick(N, 1000)

    def fullspec(a):
        nd = a.ndim
        return pl.BlockSpec(a.shape, lambda i, _n=nd: (0,) * _n)

    in_specs = [pl.BlockSpec((blk, HID), lambda i: (i, 0)),
                fullspec(ps)] + [fullspec(w) for w in wlist]
    return pl.pallas_call(
        _heads_body,
        grid=(N // blk,),
        in_specs=in_specs,
        out_specs=[pl.BlockSpec((1, 1000), lambda i: (0, 0)),
                   pl.BlockSpec((1, 1), lambda i: (0, 0))],
        out_shape=[jax.ShapeDtypeStruct((1, 1000), jnp.float32),
                   jax.ShapeDtypeStruct((1, 1), jnp.float32)],
        scratch_shapes=[pltpu.VMEM((1, HID), jnp.float32),
                        pltpu.VMEM((1, HID), jnp.float32)],
    )(x, ps, *wlist)


_NW = 32
_GC = 80


def _mesh():
    return plsc.VectorSubcoreMesh(core_axis_name="c", subcore_axis_name="s")


def _sc_gather2(q, kv, src, dst):
    epw = E // _NW

    @functools.partial(
        pl.kernel,
        mesh=_mesh(),
        out_type=[jax.ShapeDtypeStruct((E, HID), jnp.float32),
                  jax.ShapeDtypeStruct((E, 2 * HID), jnp.float32)],
        scratch_types=[
            pltpu.VMEM((_GC,), jnp.int32),
            pltpu.VMEM((_GC,), jnp.int32),
            pltpu.VMEM((_GC, HID), jnp.float32),
            pltpu.VMEM((_GC, 2 * HID), jnp.float32),
            pltpu.SemaphoreType.DMA,
        ],
    )
    def body(q_hbm, kv_hbm, src_hbm, dst_hbm,
             qd_hbm, kvs_hbm, srcv, dstv, qbuf, kvbuf, sem):
        c = lax.axis_index("c")
        s = lax.axis_index("s")
        wid = s * 2 + c
        base0 = wid * epw

        def step(i, _):
            base = base0 + i * _GC
            pltpu.sync_copy(src_hbm.at[pl.ds(base, _GC)], srcv)
            pltpu.sync_copy(dst_hbm.at[pl.ds(base, _GC)], dstv)
            d1 = pltpu.async_copy(q_hbm.at[dstv], qbuf, sem)
            d2 = pltpu.async_copy(kv_hbm.at[srcv], kvbuf, sem)
            d1.wait()
            d2.wait()
            pltpu.sync_copy(qbuf, qd_hbm.at[pl.ds(base, _GC)])
            pltpu.sync_copy(kvbuf, kvs_hbm.at[pl.ds(base, _GC)])
            return 0

        lax.fori_loop(0, epw // _GC, step, 0)

    return body(q, kv, src, dst)


def _sc_scatter(msgT, wT, dst):
    npad = 10240
    EC = 1280
    iters = E // EC

    @functools.partial(
        pl.kernel,
        mesh=_mesh(),
        out_type=[jax.ShapeDtypeStruct((HID, npad), jnp.float32),
                  jax.ShapeDtypeStruct((32, npad), jnp.float32)],
        scratch_types=[
            pltpu.VMEM((EC,), jnp.int32),
            pltpu.VMEM((8, EC), jnp.float32),
            pltpu.VMEM((8, npad), jnp.float32),
            pltpu.VMEM((npad,), jnp.float32),
            pltpu.SemaphoreType.DMA,
            pltpu.SemaphoreType.DMA,
        ],
        compiler_params=pltpu.CompilerParams(needs_layout_passes=False),
    )
    def body(msgT_hbm, wT_hbm, dst_hbm, numT_hbm, denT_hbm,
             dstv, mbuf, acc, dacc, sem1, sem2):
        c = lax.axis_index("c")
        s = lax.axis_index("s")
        t = s * 2 + c
        zeros = jnp.zeros((16,), jnp.float32)

        def zstep(i, _):
            for r in range(8):
                acc[r, pl.ds(i * 16, 16)] = zeros
            dacc[pl.ds(i * 16, 16)] = zeros
            return 0

        lax.fori_loop(0, npad // 16, zstep, 0)

        def step(i, _):
            base = i * EC
            d1 = pltpu.async_copy(dst_hbm.at[pl.ds(base, EC)], dstv, sem1)
            d2 = pltpu.async_copy(
                msgT_hbm.at[pl.ds(t * 8, 8), pl.ds(base, EC)], mbuf, sem2)
            d1.wait()
            d2.wait()
            for g in range(EC // 16):
                dvec = dstv[pl.ds(g * 16, 16)]
                for r in range(8):
                    vals = mbuf[r, pl.ds(g * 16, 16)]
                    plsc.addupdate_scatter(
                        acc, [jnp.full((16,), r, jnp.int32), dvec], vals)
            return 0

        lax.fori_loop(0, iters, step, 0)
        pltpu.sync_copy(acc, numT_hbm.at[pl.ds(t * 8, 8)])

        def dstep(i, _):
            base = i * EC
            d1 = pltpu.async_copy(dst_hbm.at[pl.ds(base, EC)], dstv, sem1)
            d2 = pltpu.async_copy(
                wT_hbm.at[pl.ds(t * 8, 8), pl.ds(base, EC)], mbuf, sem2)
            d1.wait()
            d2.wait()
            for g in range(EC // 16):
                dvec = dstv[pl.ds(g * 16, 16)]
                vals = mbuf[0, pl.ds(g * 16, 16)]
                plsc.addupdate_scatter(dacc, [dvec], vals)
            return 0

        lax.fori_loop(0, jnp.where(t < 4, iters, 0), dstep, 0)

        def dwb(i, _):
            acc[0, pl.ds(i * 16, 16)] = dacc[pl.ds(i * 16, 16)]
            for r in range(1, 8):
                acc[r, pl.ds(i * 16, 16)] = zeros
            return 0

        lax.fori_loop(0, jnp.where(t < 4, npad // 16, 0), dwb, 0)

        def dwb2(i, _):
            pltpu.sync_copy(acc, denT_hbm.at[pl.ds(t * 8, 8)])
            return 0

        lax.fori_loop(0, jnp.where(t < 4, 1, 0), dwb2, 0)
        return None

    return body(msgT, wT, dst)


def kernel(x, edge_index, edge_attr, private_state, params):
    f32 = jnp.float32
    src = edge_index[0].astype(jnp.int32)
    dst = edge_index[1].astype(jnp.int32)

    eye4 = jnp.eye(4, dtype=f32)
    bsel = jnp.zeros((HID, 8), f32).at[:, :4].set(
        jnp.repeat(eye4, CH, axis=0) * (1.0 / np.sqrt(CH)))
    s8 = jnp.zeros((8, HID), f32).at[:4, :].set(jnp.repeat(eye4, CH, axis=1))
    selw = jnp.zeros((32, 8), f32).at[jnp.arange(4) * 8, jnp.arange(4)].set(1.0)
    sel = jnp.zeros((HID, 32), f32).at[jnp.arange(HID),
                                       (jnp.arange(HID) // CH) * 8].set(1.0)

    def r2(a):
        return a.reshape(1, -1)

    ne, ee = params["node_enc"], params["edge_enc"]
    xh = _encoder(x, ne["lin"]["W"], r2(ne["lin"]["b"]),
                  r2(ne["ln"]["g"]), r2(ne["ln"]["b"]), blk=_pick(N, 1000))
    if N == 10000:
        xh = jnp.zeros((10240, HID), f32).at[:N].set(xh)
    ea = _encoder(edge_attr, ee["lin"]["W"], r2(ee["lin"]["b"]),
                  r2(ee["ln"]["g"]), r2(ee["ln"]["b"]), blk=_pick(E, 2000))

    for lp in params["gnn"]:
        wkv = jnp.concatenate([lp["k"]["W"], lp["v"]["W"]], axis=1)
        bkv = jnp.concatenate([lp["k"]["b"], lp["v"]["b"]]).reshape(1, -1)
        q, kv = _qkv(xh, lp["q"]["W"], r2(lp["q"]["b"]), wkv, bkv)
        qd, kvs = _sc_gather2(q, kv, src, dst)
        alpha, amax = _alpha(qd, kvs, ea, lp["e"]["W"], bsel)
        msgT, wT = _msg(alpha, amax, kvs, ea, lp["e"]["W"], s8, selw)
        numT, denT = _sc_scatter(msgT, wT, dst)
        xh = _update(xh, numT, denT, sel, lp["skip"]["W"], r2(lp["skip"]["b"]),
                     r2(lp["ln"]["g"]), r2(lp["ln"]["b"]))

    pp, fp = params["priv"], params["fusion"]
    fwa = fp["lin1"]["W"][:HID]
    fwb = fp["lin1"]["W"][HID:2 * HID]
    fwc = fp["lin1"]["W"][2 * HID:]
    wlist = [
        pp["lin1"]["W"], r2(pp["lin1"]["b"]), r2(pp["ln1"]["g"]), r2(pp["ln1"]["b"]),
        pp["lin2"]["W"], r2(pp["lin2"]["b"]), r2(pp["ln2"]["g"]), r2(pp["ln2"]["b"]),
        fwa, fwb, fwc, r2(fp["lin1"]["b"]), r2(fp["ln1"]["g"]), r2(fp["ln1"]["b"]),
        fp["lin2"]["W"], r2(fp["lin2"]["b"]), r2(fp["ln2"]["g"]), r2(fp["ln2"]["b"]),
        params["policy"][0]["W"], r2(params["policy"][0]["b"]),
        params["policy"][1]["W"], r2(params["policy"][1]["b"]),
        params["policy"][2]["W"], r2(params["policy"][2]["b"]),
        params["value"][0]["W"], r2(params["value"][0]["b"]),
        params["value"][1]["W"], r2(params["value"][1]["b"]),
        params["value"][2]["W"], r2(params["value"][2]["b"]),
    ]
    pol, val = _heads(xh[:N], r2(private_state), wlist)
    return pol, val
